# baseline (speedup 1.0000x reference)
import functools

import jax
import jax.numpy as jnp
from jax import lax
from jax.experimental import pallas as pl
from jax.experimental.pallas import tpu as pltpu
from jax.experimental.pallas import tpu_sc as plsc

N = 10000
E = 320000
NUM_TYPES = 32
D = 128
HID = 256
NCLS = 6
EPS = 1e-5

NC, NS, LANES = 2, 16, 16
NTILES = NC * NS
NPAD = 10240
HALF = NPAD // 2
RPAD = 5248
DUMP = 5184
ROWS_PER_TILE = NPAD // NTILES
RSUB = RPAD // NS
CHUNK = 128
NSTAGE = 2
SCHUNK = 80
NCHUNK = NSTAGE * SCHUNK
EPT = CHUNK * NCHUNK
EPAD = NS * EPT
NBUF = 4

_f32 = jnp.float32


@functools.lru_cache(maxsize=None)
def _get_mesh():
    return plsc.VectorSubcoreMesh(core_axis_name="c", subcore_axis_name="s")


@functools.lru_cache(maxsize=None)
def _get_embed_sc():
    @functools.partial(
        pl.kernel,
        out_type=jax.ShapeDtypeStruct((NPAD, D), _f32),
        mesh=_get_mesh(),
        scratch_types=[
            pltpu.VMEM((4, 80), jnp.int32),
            pltpu.VMEM((ROWS_PER_TILE, D), _f32),
            pltpu.SemaphoreType.DMA,
        ],
    )
    def _embed_sc(emb, nf, out, idx_v, rows_v, sem):
        c = lax.axis_index("c")
        s = lax.axis_index("s")
        wid = s * NC + c
        pltpu.sync_copy(nf.at[wid], idx_v)
        for j in range(4):
            pltpu.async_copy(emb.at[idx_v.at[j]],
                             rows_v.at[pl.ds(j * 80, 80)], sem).wait()
        pltpu.sync_copy(rows_v, out.at[pl.ds(wid * ROWS_PER_TILE, ROWS_PER_TILE)])

    return _embed_sc


@functools.lru_cache(maxsize=None)
def _get_agg_sc():
    @functools.partial(
        pl.kernel,
        out_type=jax.ShapeDtypeStruct((NC, RPAD, D), _f32),
        mesh=_get_mesh(),
        scratch_types=[
            pltpu.VMEM((SCHUNK, CHUNK), jnp.int32),
            pltpu.VMEM((SCHUNK, CHUNK), jnp.int32),
            pltpu.VMEM((NBUF, CHUNK, D), _f32),
            pltpu.VMEM_SHARED((RPAD, D), _f32),
            pltpu.SemaphoreType.DMA,
            pltpu.SemaphoreType.DMA,
        ],
    )
    def _agg_sc(u, srci, dsti, zer, agg_out,
                src_v, dst_v, buf, aggs, gsem, ssem):
        c = lax.axis_index("c")
        s = lax.axis_index("s")
        rows = pl.ds(s * RSUB, RSUB)
        pltpu.sync_copy(zer.at[rows], aggs.at[rows])
        plsc.subcore_barrier()

        for st in range(NSTAGE):
            pltpu.sync_copy(srci.at[s, st], src_v)
            pltpu.sync_copy(dsti.at[c, s, st], dst_v)
            for j in range(NBUF - 1):
                pltpu.async_copy(u.at[src_v.at[j]], buf.at[j], gsem)

            def step(jj, carry):
                for b in range(NBUF):
                    j = jj * NBUF + b
                    pltpu.make_async_copy(
                        u.at[src_v.at[j]], buf.at[b], gsem).wait()
                    pltpu.async_copy(
                        buf.at[b], aggs.at[dst_v.at[j]], ssem, add=True)

                    @pl.when(j + NBUF - 1 < SCHUNK)
                    def _():
                        nb = (b + NBUF - 1) % NBUF
                        @pl.when(j >= 1)
                        def _():
                            pltpu.make_async_copy(
                                buf.at[nb], aggs.at[dst_v.at[j]], ssem).wait()
                        pltpu.async_copy(
                            u.at[src_v.at[j + NBUF - 1]], buf.at[nb], gsem)
                return carry

            lax.fori_loop(0, SCHUNK // NBUF, step, 0)
            for _ in range(NBUF):
                pltpu.make_async_copy(
                    buf.at[0], aggs.at[dst_v.at[SCHUNK - 1]], ssem).wait()
        plsc.subcore_barrier()
        pltpu.sync_copy(aggs.at[rows], agg_out.at[c, rows])

    return _agg_sc


@functools.lru_cache(maxsize=None)
def _get_deg_sc():
    @functools.partial(
        pl.kernel,
        out_type=jax.ShapeDtypeStruct((NC, RPAD, D), _f32),
        mesh=_get_mesh(),
        scratch_types=[
            pltpu.VMEM((SCHUNK, CHUNK), jnp.int32),
            pltpu.VMEM((CHUNK, D), _f32),
            pltpu.VMEM_SHARED((RPAD, D), _f32),
            pltpu.SemaphoreType.DMA,
        ],
    )
    def _deg_sc(dsti, zer, ones, deg_out, dst_v, ones_v, degs, dsem):
        c = lax.axis_index("c")
        s = lax.axis_index("s")
        pltpu.sync_copy(ones, ones_v)
        rows = pl.ds(s * RSUB, RSUB)
        pltpu.sync_copy(zer.at[rows], degs.at[rows])
        plsc.subcore_barrier()

        for st in range(NSTAGE):
            pltpu.sync_copy(dsti.at[c, s, st], dst_v)

            def step(j, carry):
                @pl.when(j >= 2)
                def _():
                    pltpu.make_async_copy(
                        ones_v, degs.at[dst_v.at[j]], dsem).wait()
                pltpu.async_copy(ones_v, degs.at[dst_v.at[j]], dsem, add=True)
                return carry

            lax.fori_loop(0, SCHUNK, step, 0)
            for _ in range(2):
                pltpu.make_async_copy(
                    ones_v, degs.at[dst_v.at[SCHUNK - 1]], dsem).wait()
        plsc.subcore_barrier()
        pltpu.sync_copy(degs.at[rows], deg_out.at[c, rows])

    return _deg_sc


BLK = 512
GRID = NPAD // BLK
RBLK = HALF // BLK


def _affine(sums_ref, gam_ref, bet_ref):
    sm = sums_ref[0:1, :]
    sq = sums_ref[1:2, :]
    mean = sm * (1.0 / N)
    var = sq * (1.0 / N) - mean * mean
    a = gam_ref[...] * lax.rsqrt(var + EPS)
    b = bet_ref[...] - mean * a
    return a, b


def _layer_body(u_ref, agg_ref, deg_ref, sums_ref, gam_ref, bet_ref,
                w1a_ref, w1b_ref, b1_ref, w2_ref, b2_ref,
                u_out, sums_out):
    i = pl.program_id(0)
    a, b = _affine(sums_ref, gam_ref, bet_ref)
    hn = u_ref[...] * a + b
    degc = deg_ref[0, :, 0:1]
    an = agg_ref[0] * a + degc * b
    z = jnp.dot(hn, w1a_ref[...], preferred_element_type=_f32)
    z = z + jnp.dot(an, w1b_ref[...], preferred_element_type=_f32)
    z = jnp.maximum(z + b1_ref[...], 0.0)
    un = jnp.dot(z, w2_ref[...], preferred_element_type=_f32) + b2_ref[...]
    u_out[...] = un

    @pl.when(i == 0)
    def _():
        sums_out[...] = jnp.zeros((8, D), _f32)

    rid = i * BLK + lax.broadcasted_iota(jnp.int32, (BLK, 1), 0)
    unm = jnp.where(rid < N, un, 0.0)
    sums_out[0:1, :] = sums_out[0:1, :] + jnp.sum(unm, 0, keepdims=True)
    sums_out[1:2, :] = sums_out[1:2, :] + jnp.sum(unm * unm, 0, keepdims=True)


def _half_map(i):
    return (i // RBLK, i % RBLK, 0)


def _layer_tc(u, agg, deg, sums, gamma, beta, w1a, w1b, b1, w2, b2):
    return pl.pallas_call(
        _layer_body,
        grid=(GRID,),
        in_specs=[
            pl.BlockSpec((BLK, D), lambda i: (i, 0)),
            pl.BlockSpec((1, BLK, D), _half_map),
            pl.BlockSpec((1, BLK, D), _half_map),
            pl.BlockSpec((8, D), lambda i: (0, 0)),
            pl.BlockSpec((1, D), lambda i: (0, 0)),
            pl.BlockSpec((1, D), lambda i: (0, 0)),
            pl.BlockSpec((D, HID), lambda i: (0, 0)),
            pl.BlockSpec((D, HID), lambda i: (0, 0)),
            pl.BlockSpec((1, HID), lambda i: (0, 0)),
            pl.BlockSpec((HID, D), lambda i: (0, 0)),
            pl.BlockSpec((1, D), lambda i: (0, 0)),
        ],
        out_specs=[
            pl.BlockSpec((BLK, D), lambda i: (i, 0)),
            pl.BlockSpec((8, D), lambda i: (0, 0)),
        ],
        out_shape=[
            jax.ShapeDtypeStruct((NPAD, D), _f32),
            jax.ShapeDtypeStruct((8, D), _f32),
        ],
    )(u, agg, deg, sums, gamma, beta, w1a, w1b, b1, w2, b2)


RBLK_O = 400
GRID_O = N // RBLK_O


def _readout_body(u_ref, sums_ref, gam_ref, bet_ref,
                  r1_ref, rb1_ref, r2_ref, rb2_ref, r3_ref, rb3_ref, out):
    a, b = _affine(sums_ref, gam_ref, bet_ref)
    hn = u_ref[...] * a + b
    x = jnp.maximum(jnp.dot(hn, r1_ref[...], preferred_element_type=_f32)
                    + rb1_ref[...], 0.0)
    x = jnp.maximum(jnp.dot(x, r2_ref[...], preferred_element_type=_f32)
                    + rb2_ref[...], 0.0)
    out[...] = jnp.dot(x, r3_ref[...], preferred_element_type=_f32) + rb3_ref[...]


def _readout_tc(u, sums, gamma, beta, r1, rb1, r2, rb2, r3, rb3):
    return pl.pallas_call(
        _readout_body,
        grid=(GRID_O,),
        in_specs=[
            pl.BlockSpec((RBLK_O, D), lambda i: (i, 0)),
            pl.BlockSpec((8, D), lambda i: (0, 0)),
            pl.BlockSpec((1, D), lambda i: (0, 0)),
            pl.BlockSpec((1, D), lambda i: (0, 0)),
            pl.BlockSpec((D, 64), lambda i: (0, 0)),
            pl.BlockSpec((1, 64), lambda i: (0, 0)),
            pl.BlockSpec((64, 32), lambda i: (0, 0)),
            pl.BlockSpec((1, 32), lambda i: (0, 0)),
            pl.BlockSpec((32, D), lambda i: (0, 0)),
            pl.BlockSpec((1, D), lambda i: (0, 0)),
        ],
        out_specs=pl.BlockSpec((RBLK_O, D), lambda i: (i, 0)),
        out_shape=jax.ShapeDtypeStruct((N, D), _f32),
    )(u, sums, gamma, beta, r1, rb1, r2, rb2, r3, rb3)


def kernel(edge_index, node_feat, edge_feat, snorm_n, snorm_e, params):
    del edge_feat, snorm_n, snorm_e
    src = edge_index[0]
    dst = edge_index[1]
    pad = EPAD - E
    src_r = jnp.concatenate(
        [src, jnp.zeros((pad,), jnp.int32)]).reshape(NS, NSTAGE, SCHUNK, CHUNK)
    dst_p = jnp.concatenate(
        [dst, N + (jnp.arange(pad, dtype=jnp.int32) % (NPAD - N))])
    dump = DUMP + (jnp.arange(EPAD, dtype=jnp.int32) & 63)
    dst_c = jnp.stack([
        jnp.where(dst_p < HALF, dst_p, dump),
        jnp.where(dst_p >= HALF, dst_p - HALF, dump),
    ]).reshape(NC, NS, NSTAGE, SCHUNK, CHUNK)
    nf_r = jnp.concatenate(
        [node_feat, jnp.zeros((NPAD - N,), jnp.int32)]).reshape(NTILES, 4, 80)
    zer = jnp.zeros((RPAD, D), _f32)
    ones128 = jnp.ones((CHUNK, D), _f32)

    h0 = _get_embed_sc()(params['embed'], nf_r)
    deg = _get_deg_sc()(dst_c, zer, ones128)

    sums = jnp.concatenate([
        jnp.zeros((1, D), _f32),
        jnp.full((1, D), N * (1.0 - EPS), _f32),
        jnp.zeros((6, D), _f32),
    ])
    gamma = jnp.ones((1, D), _f32)
    beta = jnp.zeros((1, D), _f32)

    u = h0
    for lp in params['layers']:
        agg = _get_agg_sc()(u, src_r, dst_c, zer)
        w1a = lp['W1'][:D]
        w1b = lp['W1'][D:]
        u, sums = _layer_tc(
            u, agg, deg, sums, gamma, beta,
            w1a, w1b, lp['b1'].reshape(1, HID),
            lp['W2'], lp['b2'].reshape(1, D))
        gamma = lp['gamma'].reshape(1, D)
        beta = lp['beta'].reshape(1, D)

    rp = params['readout']
    r3 = jnp.zeros((32, D), _f32).at[:, :NCLS].set(rp[2]['W'])
    rb3 = jnp.zeros((1, D), _f32).at[:, :NCLS].set(rp[2]['b'].reshape(1, NCLS))
    out = _readout_tc(
        u, sums, gamma, beta,
        rp[0]['W'], rp[0]['b'].reshape(1, 64),
        rp[1]['W'], rp[1]['b'].reshape(1, 32),
        r3, rb3)
    return out[:, :NCLS]


# speedup vs baseline: 1.4390x; 1.4390x over previous
import functools

import jax
import jax.numpy as jnp
from jax import lax
from jax.experimental import pallas as pl
from jax.experimental.pallas import tpu as pltpu
from jax.experimental.pallas import tpu_sc as plsc

N = 10000
E = 320000
NUM_TYPES = 32
D = 128
HID = 256
NCLS = 6
EPS = 1e-5

NC, NS, LANES = 2, 16, 16
NTILES = NC * NS
NPAD = 10240
HALF = NPAD // 2
RPAD = 5248
DUMP = 5184
ROWS_PER_TILE = NPAD // NTILES
RSUB = RPAD // NS
CHUNK = 128
NCHUNK = 158
EPT = CHUNK * NCHUNK
EPAD = NS * EPT

_f32 = jnp.float32


@functools.lru_cache(maxsize=None)
def _get_mesh():
    return plsc.VectorSubcoreMesh(core_axis_name="c", subcore_axis_name="s")


@functools.lru_cache(maxsize=None)
def _get_embed_sc():
    @functools.partial(
        pl.kernel,
        out_type=jax.ShapeDtypeStruct((NPAD, D), _f32),
        mesh=_get_mesh(),
        scratch_types=[
            pltpu.VMEM((4, 80), jnp.int32),
            pltpu.VMEM((ROWS_PER_TILE, D), _f32),
            pltpu.SemaphoreType.DMA,
        ],
    )
    def _embed_sc(emb, nf, out, idx_v, rows_v, sem):
        c = lax.axis_index("c")
        s = lax.axis_index("s")
        wid = s * NC + c
        pltpu.sync_copy(nf.at[wid], idx_v)
        for j in range(4):
            pltpu.async_copy(emb.at[idx_v.at[j]],
                             rows_v.at[pl.ds(j * 80, 80)], sem).wait()
        pltpu.sync_copy(rows_v, out.at[pl.ds(wid * ROWS_PER_TILE, ROWS_PER_TILE)])

    return _embed_sc


@functools.lru_cache(maxsize=None)
def _get_agg_sc():
    @functools.partial(
        pl.kernel,
        out_type=jax.ShapeDtypeStruct((NC, RPAD, D), _f32),
        mesh=_get_mesh(),
        scratch_types=[
            pltpu.VMEM((NCHUNK, CHUNK), jnp.int32),
            pltpu.VMEM((NCHUNK, CHUNK), jnp.int32),
            pltpu.VMEM((2, CHUNK, D), _f32),
            pltpu.VMEM_SHARED((RPAD, D), _f32),
            pltpu.SemaphoreType.DMA,
            pltpu.SemaphoreType.DMA,
        ],
    )
    def _agg_sc(u, srci, dsti, zer, agg_out,
                src_v, dst_v, buf, aggs, gsem, ssem):
        c = lax.axis_index("c")
        s = lax.axis_index("s")
        pltpu.sync_copy(srci.at[s], src_v)
        pltpu.sync_copy(dsti.at[c, s], dst_v)
        rows = pl.ds(s * RSUB, RSUB)
        pltpu.sync_copy(zer.at[rows], aggs.at[rows])
        plsc.subcore_barrier()

        pltpu.async_copy(u.at[src_v.at[0]], buf.at[0], gsem)

        def step(jj, carry):
            for b in range(2):
                j = jj * 2 + b
                pltpu.make_async_copy(u.at[src_v.at[j]], buf.at[b], gsem).wait()
                pltpu.async_copy(buf.at[b], aggs.at[dst_v.at[j]], ssem, add=True)
                @pl.when(j >= 1)
                def _():
                    pltpu.make_async_copy(
                        buf.at[1 - b], aggs.at[dst_v.at[j]], ssem).wait()

                @pl.when(j + 1 < NCHUNK)
                def _():
                    pltpu.async_copy(u.at[src_v.at[j + 1]], buf.at[1 - b], gsem)
            return carry

        lax.fori_loop(0, NCHUNK // 2, step, 0)
        pltpu.make_async_copy(
            buf.at[1], aggs.at[dst_v.at[NCHUNK - 1]], ssem).wait()
        plsc.subcore_barrier()
        pltpu.sync_copy(aggs.at[rows], agg_out.at[c, rows])

    return _agg_sc


@functools.lru_cache(maxsize=None)
def _get_deg_sc():
    @functools.partial(
        pl.kernel,
        out_type=jax.ShapeDtypeStruct((NC, RPAD, D), _f32),
        mesh=_get_mesh(),
        scratch_types=[
            pltpu.VMEM((NCHUNK, CHUNK), jnp.int32),
            pltpu.VMEM((CHUNK, D), _f32),
            pltpu.VMEM_SHARED((RPAD, D), _f32),
            pltpu.SemaphoreType.DMA,
        ],
    )
    def _deg_sc(dsti, zer, ones, deg_out, dst_v, ones_v, degs, dsem):
        c = lax.axis_index("c")
        s = lax.axis_index("s")
        pltpu.sync_copy(dsti.at[c, s], dst_v)
        pltpu.sync_copy(ones, ones_v)
        rows = pl.ds(s * RSUB, RSUB)
        pltpu.sync_copy(zer.at[rows], degs.at[rows])
        plsc.subcore_barrier()

        def step(j, carry):
            @pl.when(j >= 2)
            def _():
                pltpu.make_async_copy(
                    ones_v, degs.at[dst_v.at[j]], dsem).wait()
            pltpu.async_copy(ones_v, degs.at[dst_v.at[j]], dsem, add=True)
            return carry

        lax.fori_loop(0, NCHUNK, step, 0)
        for _ in range(2):
            pltpu.make_async_copy(
                ones_v, degs.at[dst_v.at[NCHUNK - 1]], dsem).wait()
        plsc.subcore_barrier()
        pltpu.sync_copy(degs.at[rows], deg_out.at[c, rows])

    return _deg_sc


BLK = 512
GRID = NPAD // BLK
RBLK = HALF // BLK


def _affine(sums_ref, gam_ref, bet_ref):
    sm = sums_ref[0:1, :]
    sq = sums_ref[1:2, :]
    mean = sm * (1.0 / N)
    var = sq * (1.0 / N) - mean * mean
    a = gam_ref[...] * lax.rsqrt(var + EPS)
    b = bet_ref[...] - mean * a
    return a, b


def _layer_body(u_ref, agg_ref, deg_ref, sums_ref, gam_ref, bet_ref,
                w1a_ref, w1b_ref, b1_ref, w2_ref, b2_ref,
                u_out, sums_out):
    i = pl.program_id(0)
    a, b = _affine(sums_ref, gam_ref, bet_ref)
    hn = u_ref[...] * a + b
    degc = deg_ref[0, :, 0:1]
    an = agg_ref[0] * a + degc * b
    z = jnp.dot(hn, w1a_ref[...], preferred_element_type=_f32)
    z = z + jnp.dot(an, w1b_ref[...], preferred_element_type=_f32)
    z = jnp.maximum(z + b1_ref[...], 0.0)
    un = jnp.dot(z, w2_ref[...], preferred_element_type=_f32) + b2_ref[...]
    u_out[...] = un

    @pl.when(i == 0)
    def _():
        sums_out[...] = jnp.zeros((8, D), _f32)

    rid = i * BLK + lax.broadcasted_iota(jnp.int32, (BLK, 1), 0)
    unm = jnp.where(rid < N, un, 0.0)
    sums_out[0:1, :] = sums_out[0:1, :] + jnp.sum(unm, 0, keepdims=True)
    sums_out[1:2, :] = sums_out[1:2, :] + jnp.sum(unm * unm, 0, keepdims=True)


def _half_map(i):
    return (i // RBLK, i % RBLK, 0)


def _layer_tc(u, agg, deg, sums, gamma, beta, w1a, w1b, b1, w2, b2):
    return pl.pallas_call(
        _layer_body,
        grid=(GRID,),
        in_specs=[
            pl.BlockSpec((BLK, D), lambda i: (i, 0)),
            pl.BlockSpec((1, BLK, D), _half_map),
            pl.BlockSpec((1, BLK, D), _half_map),
            pl.BlockSpec((8, D), lambda i: (0, 0)),
            pl.BlockSpec((1, D), lambda i: (0, 0)),
            pl.BlockSpec((1, D), lambda i: (0, 0)),
            pl.BlockSpec((D, HID), lambda i: (0, 0)),
            pl.BlockSpec((D, HID), lambda i: (0, 0)),
            pl.BlockSpec((1, HID), lambda i: (0, 0)),
            pl.BlockSpec((HID, D), lambda i: (0, 0)),
            pl.BlockSpec((1, D), lambda i: (0, 0)),
        ],
        out_specs=[
            pl.BlockSpec((BLK, D), lambda i: (i, 0)),
            pl.BlockSpec((8, D), lambda i: (0, 0)),
        ],
        out_shape=[
            jax.ShapeDtypeStruct((NPAD, D), _f32),
            jax.ShapeDtypeStruct((8, D), _f32),
        ],
    )(u, agg, deg, sums, gamma, beta, w1a, w1b, b1, w2, b2)


RBLK_O = 400
GRID_O = N // RBLK_O


def _readout_body(u_ref, sums_ref, gam_ref, bet_ref,
                  r1_ref, rb1_ref, r2_ref, rb2_ref, r3_ref, rb3_ref, out):
    a, b = _affine(sums_ref, gam_ref, bet_ref)
    hn = u_ref[...] * a + b
    x = jnp.maximum(jnp.dot(hn, r1_ref[...], preferred_element_type=_f32)
                    + rb1_ref[...], 0.0)
    x = jnp.maximum(jnp.dot(x, r2_ref[...], preferred_element_type=_f32)
                    + rb2_ref[...], 0.0)
    out[...] = jnp.dot(x, r3_ref[...], preferred_element_type=_f32) + rb3_ref[...]


def _readout_tc(u, sums, gamma, beta, r1, rb1, r2, rb2, r3, rb3):
    return pl.pallas_call(
        _readout_body,
        grid=(GRID_O,),
        in_specs=[
            pl.BlockSpec((RBLK_O, D), lambda i: (i, 0)),
            pl.BlockSpec((8, D), lambda i: (0, 0)),
            pl.BlockSpec((1, D), lambda i: (0, 0)),
            pl.BlockSpec((1, D), lambda i: (0, 0)),
            pl.BlockSpec((D, 64), lambda i: (0, 0)),
            pl.BlockSpec((1, 64), lambda i: (0, 0)),
            pl.BlockSpec((64, 32), lambda i: (0, 0)),
            pl.BlockSpec((1, 32), lambda i: (0, 0)),
            pl.BlockSpec((32, D), lambda i: (0, 0)),
            pl.BlockSpec((1, D), lambda i: (0, 0)),
        ],
        out_specs=pl.BlockSpec((RBLK_O, D), lambda i: (i, 0)),
        out_shape=jax.ShapeDtypeStruct((N, D), _f32),
    )(u, sums, gamma, beta, r1, rb1, r2, rb2, r3, rb3)


def kernel(edge_index, node_feat, edge_feat, snorm_n, snorm_e, params):
    del edge_feat, snorm_n, snorm_e
    src = edge_index[0]
    dst = edge_index[1]
    pad = EPAD - E
    src_r = jnp.concatenate(
        [src, jnp.zeros((pad,), jnp.int32)]).reshape(NS, NCHUNK, CHUNK)
    dst_p = jnp.concatenate([dst, jnp.full((pad,), NPAD - 1, jnp.int32)])
    dump = DUMP + (jnp.arange(EPAD, dtype=jnp.int32) & 63)
    dst_c = jnp.stack([
        jnp.where(dst_p < HALF, dst_p, dump),
        jnp.where(dst_p >= HALF, dst_p - HALF, dump),
    ]).reshape(NC, NS, NCHUNK, CHUNK)
    nf_r = jnp.concatenate(
        [node_feat, jnp.zeros((NPAD - N,), jnp.int32)]).reshape(NTILES, 4, 80)
    zer = jnp.zeros((RPAD, D), _f32)
    ones128 = jnp.ones((CHUNK, D), _f32)

    h0 = _get_embed_sc()(params['embed'], nf_r)
    deg = _get_deg_sc()(dst_c, zer, ones128)

    sums = jnp.concatenate([
        jnp.zeros((1, D), _f32),
        jnp.full((1, D), N * (1.0 - EPS), _f32),
        jnp.zeros((6, D), _f32),
    ])
    gamma = jnp.ones((1, D), _f32)
    beta = jnp.zeros((1, D), _f32)

    u = h0
    for lp in params['layers']:
        agg = _get_agg_sc()(u, src_r, dst_c, zer)
        w1a = lp['W1'][:D]
        w1b = lp['W1'][D:]
        u, sums = _layer_tc(
            u, agg, deg, sums, gamma, beta,
            w1a, w1b, lp['b1'].reshape(1, HID),
            lp['W2'], lp['b2'].reshape(1, D))
        gamma = lp['gamma'].reshape(1, D)
        beta = lp['beta'].reshape(1, D)

    rp = params['readout']
    r3 = jnp.zeros((32, D), _f32).at[:, :NCLS].set(rp[2]['W'])
    rb3 = jnp.zeros((1, D), _f32).at[:, :NCLS].set(rp[2]['b'].reshape(1, NCLS))
    out = _readout_tc(
        u, sums, gamma, beta,
        rp[0]['W'], rp[0]['b'].reshape(1, 64),
        rp[1]['W'], rp[1]['b'].reshape(1, 32),
        r3, rb3)
    return out[:, :NCLS]


# speedup vs baseline: 1.4405x; 1.0011x over previous
import functools

import jax
import jax.numpy as jnp
from jax import lax
from jax.experimental import pallas as pl
from jax.experimental.pallas import tpu as pltpu
from jax.experimental.pallas import tpu_sc as plsc

N = 10000
E = 320000
NUM_TYPES = 32
D = 128
HID = 256
NCLS = 6
EPS = 1e-5

NC, NS, LANES = 2, 16, 16
NTILES = NC * NS
NPAD = 10240
HALF = NPAD // 2
RPAD = 5376
DUMP = 5184
ROWS_PER_TILE = NPAD // NTILES
RSUB = RPAD // NS
CHUNK = 128
NCHUNK = 158
EPT = CHUNK * NCHUNK
EPAD = NS * EPT

_f32 = jnp.float32


@functools.lru_cache(maxsize=None)
def _get_mesh():
    return plsc.VectorSubcoreMesh(core_axis_name="c", subcore_axis_name="s")


@functools.lru_cache(maxsize=None)
def _get_embed_sc():
    @functools.partial(
        pl.kernel,
        out_type=jax.ShapeDtypeStruct((NPAD, D), _f32),
        mesh=_get_mesh(),
        scratch_types=[
            pltpu.VMEM((4, 80), jnp.int32),
            pltpu.VMEM((ROWS_PER_TILE, D), _f32),
            pltpu.SemaphoreType.DMA,
        ],
    )
    def _embed_sc(emb, nf, out, idx_v, rows_v, sem):
        c = lax.axis_index("c")
        s = lax.axis_index("s")
        wid = s * NC + c
        pltpu.sync_copy(nf.at[wid], idx_v)
        for j in range(4):
            pltpu.async_copy(emb.at[idx_v.at[j]],
                             rows_v.at[pl.ds(j * 80, 80)], sem).wait()
        pltpu.sync_copy(rows_v, out.at[pl.ds(wid * ROWS_PER_TILE, ROWS_PER_TILE)])

    return _embed_sc


@functools.lru_cache(maxsize=None)
def _get_agg_sc():
    @functools.partial(
        pl.kernel,
        out_type=jax.ShapeDtypeStruct((NC, RPAD, D), _f32),
        mesh=_get_mesh(),
        scratch_types=[
            pltpu.VMEM((NCHUNK, CHUNK), jnp.int32),
            pltpu.VMEM((NCHUNK, CHUNK), jnp.int32),
            pltpu.VMEM((2, CHUNK, D), _f32),
            pltpu.VMEM_SHARED((RPAD, D), _f32),
            pltpu.SemaphoreType.DMA,
            pltpu.SemaphoreType.DMA,
        ],
    )
    def _agg_sc(u, srci, dsti, zer, agg_out,
                src_v, dst_v, buf, aggs, gsem, ssem):
        c = lax.axis_index("c")
        s = lax.axis_index("s")
        pltpu.sync_copy(srci.at[s], src_v)
        pltpu.sync_copy(dsti.at[c, s], dst_v)
        rows = pl.ds(s * RSUB, RSUB)
        pltpu.sync_copy(zer.at[rows], aggs.at[rows])
        plsc.subcore_barrier()

        pltpu.async_copy(u.at[src_v.at[0]], buf.at[0], gsem)

        def step(jj, carry):
            for b in range(2):
                j = jj * 2 + b
                pltpu.make_async_copy(u.at[src_v.at[j]], buf.at[b], gsem).wait()
                pltpu.async_copy(buf.at[b], aggs.at[dst_v.at[j]], ssem, add=True)
                @pl.when(j >= 1)
                def _():
                    pltpu.make_async_copy(
                        buf.at[1 - b], aggs.at[dst_v.at[j]], ssem).wait()

                @pl.when(j + 1 < NCHUNK)
                def _():
                    pltpu.async_copy(u.at[src_v.at[j + 1]], buf.at[1 - b], gsem)
            return carry

        lax.fori_loop(0, NCHUNK // 2, step, 0)
        pltpu.make_async_copy(
            buf.at[1], aggs.at[dst_v.at[NCHUNK - 1]], ssem).wait()
        plsc.subcore_barrier()
        pltpu.sync_copy(aggs.at[rows], agg_out.at[c, rows])

    return _agg_sc


@functools.lru_cache(maxsize=None)
def _get_deg_sc():
    @functools.partial(
        pl.kernel,
        out_type=jax.ShapeDtypeStruct((NC, RPAD, D), _f32),
        mesh=_get_mesh(),
        scratch_types=[
            pltpu.VMEM((NCHUNK, CHUNK), jnp.int32),
            pltpu.VMEM((CHUNK, D), _f32),
            pltpu.VMEM_SHARED((RPAD, D), _f32),
            pltpu.SemaphoreType.DMA,
        ],
    )
    def _deg_sc(dsti, zer, ones, deg_out, dst_v, ones_v, degs, dsem):
        c = lax.axis_index("c")
        s = lax.axis_index("s")
        pltpu.sync_copy(dsti.at[c, s], dst_v)
        pltpu.sync_copy(ones, ones_v)
        rows = pl.ds(s * RSUB, RSUB)
        pltpu.sync_copy(zer.at[rows], degs.at[rows])
        plsc.subcore_barrier()

        def step(j, carry):
            @pl.when(j >= 2)
            def _():
                pltpu.make_async_copy(
                    ones_v, degs.at[dst_v.at[j]], dsem).wait()
            pltpu.async_copy(ones_v, degs.at[dst_v.at[j]], dsem, add=True)
            return carry

        lax.fori_loop(0, NCHUNK, step, 0)
        for _ in range(2):
            pltpu.make_async_copy(
                ones_v, degs.at[dst_v.at[NCHUNK - 1]], dsem).wait()
        plsc.subcore_barrier()
        pltpu.sync_copy(degs.at[rows], deg_out.at[c, rows])

    return _deg_sc


BLK = 512
GRID = NPAD // BLK
RBLK = HALF // BLK


def _affine(sums_ref, gam_ref, bet_ref):
    sm = sums_ref[0:1, :]
    sq = sums_ref[1:2, :]
    mean = sm * (1.0 / N)
    var = sq * (1.0 / N) - mean * mean
    a = gam_ref[...] * lax.rsqrt(var + EPS)
    b = bet_ref[...] - mean * a
    return a, b


def _layer_body(u_ref, agg_ref, deg_ref, sums_ref, gam_ref, bet_ref,
                w1a_ref, w1b_ref, b1_ref, w2_ref, b2_ref,
                u_out, sums_out):
    i = pl.program_id(0)
    a, b = _affine(sums_ref, gam_ref, bet_ref)
    hn = u_ref[...] * a + b
    degc = deg_ref[0, :, 0:1]
    an = agg_ref[0] * a + degc * b
    z = jnp.dot(hn, w1a_ref[...], preferred_element_type=_f32)
    z = z + jnp.dot(an, w1b_ref[...], preferred_element_type=_f32)
    z = jnp.maximum(z + b1_ref[...], 0.0)
    un = jnp.dot(z, w2_ref[...], preferred_element_type=_f32) + b2_ref[...]
    u_out[...] = un

    @pl.when(i == 0)
    def _():
        sums_out[...] = jnp.zeros((8, D), _f32)

    rid = i * BLK + lax.broadcasted_iota(jnp.int32, (BLK, 1), 0)
    unm = jnp.where(rid < N, un, 0.0)
    sums_out[0:1, :] = sums_out[0:1, :] + jnp.sum(unm, 0, keepdims=True)
    sums_out[1:2, :] = sums_out[1:2, :] + jnp.sum(unm * unm, 0, keepdims=True)


def _half_map(i):
    return (i // RBLK, i % RBLK, 0)


def _layer_tc(u, agg, deg, sums, gamma, beta, w1a, w1b, b1, w2, b2):
    return pl.pallas_call(
        _layer_body,
        grid=(GRID,),
        in_specs=[
            pl.BlockSpec((BLK, D), lambda i: (i, 0)),
            pl.BlockSpec((1, BLK, D), _half_map),
            pl.BlockSpec((1, BLK, D), _half_map),
            pl.BlockSpec((8, D), lambda i: (0, 0)),
            pl.BlockSpec((1, D), lambda i: (0, 0)),
            pl.BlockSpec((1, D), lambda i: (0, 0)),
            pl.BlockSpec((D, HID), lambda i: (0, 0)),
            pl.BlockSpec((D, HID), lambda i: (0, 0)),
            pl.BlockSpec((1, HID), lambda i: (0, 0)),
            pl.BlockSpec((HID, D), lambda i: (0, 0)),
            pl.BlockSpec((1, D), lambda i: (0, 0)),
        ],
        out_specs=[
            pl.BlockSpec((BLK, D), lambda i: (i, 0)),
            pl.BlockSpec((8, D), lambda i: (0, 0)),
        ],
        out_shape=[
            jax.ShapeDtypeStruct((NPAD, D), _f32),
            jax.ShapeDtypeStruct((8, D), _f32),
        ],
    )(u, agg, deg, sums, gamma, beta, w1a, w1b, b1, w2, b2)


RBLK_O = 400
GRID_O = N // RBLK_O


def _readout_body(u_ref, sums_ref, gam_ref, bet_ref,
                  r1_ref, rb1_ref, r2_ref, rb2_ref, r3_ref, rb3_ref, out):
    a, b = _affine(sums_ref, gam_ref, bet_ref)
    hn = u_ref[...] * a + b
    x = jnp.maximum(jnp.dot(hn, r1_ref[...], preferred_element_type=_f32)
                    + rb1_ref[...], 0.0)
    x = jnp.maximum(jnp.dot(x, r2_ref[...], preferred_element_type=_f32)
                    + rb2_ref[...], 0.0)
    out[...] = jnp.dot(x, r3_ref[...], preferred_element_type=_f32) + rb3_ref[...]


def _readout_tc(u, sums, gamma, beta, r1, rb1, r2, rb2, r3, rb3):
    return pl.pallas_call(
        _readout_body,
        grid=(GRID_O,),
        in_specs=[
            pl.BlockSpec((RBLK_O, D), lambda i: (i, 0)),
            pl.BlockSpec((8, D), lambda i: (0, 0)),
            pl.BlockSpec((1, D), lambda i: (0, 0)),
            pl.BlockSpec((1, D), lambda i: (0, 0)),
            pl.BlockSpec((D, 64), lambda i: (0, 0)),
            pl.BlockSpec((1, 64), lambda i: (0, 0)),
            pl.BlockSpec((64, 32), lambda i: (0, 0)),
            pl.BlockSpec((1, 32), lambda i: (0, 0)),
            pl.BlockSpec((32, D), lambda i: (0, 0)),
            pl.BlockSpec((1, D), lambda i: (0, 0)),
        ],
        out_specs=pl.BlockSpec((RBLK_O, D), lambda i: (i, 0)),
        out_shape=jax.ShapeDtypeStruct((N, D), _f32),
    )(u, sums, gamma, beta, r1, rb1, r2, rb2, r3, rb3)


def kernel(edge_index, node_feat, edge_feat, snorm_n, snorm_e, params):
    del edge_feat, snorm_n, snorm_e
    src = edge_index[0]
    dst = edge_index[1]
    pad = EPAD - E
    src_r = jnp.concatenate(
        [src, jnp.zeros((pad,), jnp.int32)]).reshape(NS, NCHUNK, CHUNK)
    dst_p = jnp.concatenate([dst, jnp.full((pad,), NPAD - 1, jnp.int32)])
    dump = DUMP + (jnp.arange(EPAD, dtype=jnp.int32) & 127)
    dst_c = jnp.stack([
        jnp.where(dst_p < HALF, dst_p, dump),
        jnp.where(dst_p >= HALF, dst_p - HALF, dump),
    ]).reshape(NC, NS, NCHUNK, CHUNK)
    nf_r = jnp.concatenate(
        [node_feat, jnp.zeros((NPAD - N,), jnp.int32)]).reshape(NTILES, 4, 80)
    zer = jnp.zeros((RPAD, D), _f32)
    ones128 = jnp.ones((CHUNK, D), _f32)

    h0 = _get_embed_sc()(params['embed'], nf_r)
    deg = _get_deg_sc()(dst_c, zer, ones128)

    sums = jnp.concatenate([
        jnp.zeros((1, D), _f32),
        jnp.full((1, D), N * (1.0 - EPS), _f32),
        jnp.zeros((6, D), _f32),
    ])
    gamma = jnp.ones((1, D), _f32)
    beta = jnp.zeros((1, D), _f32)

    u = h0
    for lp in params['layers']:
        agg = _get_agg_sc()(u, src_r, dst_c, zer)
        w1a = lp['W1'][:D]
        w1b = lp['W1'][D:]
        u, sums = _layer_tc(
            u, agg, deg, sums, gamma, beta,
            w1a, w1b, lp['b1'].reshape(1, HID),
            lp['W2'], lp['b2'].reshape(1, D))
        gamma = lp['gamma'].reshape(1, D)
        beta = lp['beta'].reshape(1, D)

    rp = params['readout']
    r3 = jnp.zeros((32, D), _f32).at[:, :NCLS].set(rp[2]['W'])
    rb3 = jnp.zeros((1, D), _f32).at[:, :NCLS].set(rp[2]['b'].reshape(1, NCLS))
    out = _readout_tc(
        u, sums, gamma, beta,
        rp[0]['W'], rp[0]['b'].reshape(1, 64),
        rp[1]['W'], rp[1]['b'].reshape(1, 32),
        r3, rb3)
    return out[:, :NCLS]
